# Initial kernel scaffold; baseline (speedup 1.0000x reference)
import jax
import jax.numpy as jnp
from jax.experimental import pallas as pl


def kernel(x, table, W, b):
    raise NotImplementedError("write your pallas kernel here")


# speedup vs baseline: 124.2355x; 124.2355x over previous
import functools

import jax
import jax.numpy as jnp
from jax import lax
from jax.experimental import pallas as pl
from jax.experimental.pallas import tpu as pltpu
from jax.experimental.pallas import tpu_sc as plsc

LANES = 16


def _tw_tc_kernel(table_ref, w_ref, b_ref, out_ref, *, inv_l):
    t = table_ref[...]
    w = w_ref[...]
    tw = jnp.dot(t, w, preferred_element_type=jnp.float32)
    out_ref[...] = (tw + b_ref[0]) * inv_l


def _make_sc_lookup(BL_flat, V_pad, B, L, n_workers, chunk_sents):
    sents_per_worker = B // n_workers
    n_chunks = sents_per_worker // chunk_sents
    blocks_per_chunk = chunk_sents // LANES
    mesh = plsc.VectorSubcoreMesh(core_axis_name="c", subcore_axis_name="s")

    @functools.partial(
        pl.kernel,
        mesh=mesh,
        out_type=jax.ShapeDtypeStruct((B,), jnp.float32),
        scratch_types=[
            pltpu.VMEM((chunk_sents * L,), jnp.int32),
            pltpu.VMEM((V_pad,), jnp.float32),
            pltpu.VMEM((sents_per_worker,), jnp.float32),
        ],
        compiler_params=pltpu.CompilerParams(needs_layout_passes=False),
    )
    def sc_lookup(x_hbm, tw_hbm, out_hbm, idx_v, tw_v, out_v):
        n_cores = 2
        wid = lax.axis_index("s") * n_cores + lax.axis_index("c")
        base_s = wid * sents_per_worker

        pltpu.sync_copy(tw_hbm, tw_v)
        iota16 = lax.iota(jnp.int32, LANES)

        for c in range(n_chunks):
            start = (base_s + c * chunk_sents) * L
            pltpu.sync_copy(x_hbm.at[pl.ds(start, chunk_sents * L)], idx_v)
            for blk in range(blocks_per_chunk):
                rowbase = (blk * LANES + iota16) * L

                def body(t, acc, rowbase=rowbase):
                    fidx = rowbase + t
                    xv = plsc.load_gather(idx_v, [fidx])
                    tv = plsc.load_gather(tw_v, [xv])
                    return acc + tv

                acc = lax.fori_loop(0, L, body, jnp.zeros((LANES,), jnp.float32))
                pred = 1.0 / (1.0 + jnp.exp(-acc))
                out_v[pl.ds(c * chunk_sents + blk * LANES, LANES)] = pred

        pltpu.sync_copy(out_v, out_hbm.at[pl.ds(base_s, sents_per_worker)])

    return sc_lookup


def kernel(x, table, W, b):
    B, L = x.shape
    V, EMB = table.shape
    V_pad = ((V + 7) // 8) * 8

    table_p = jnp.pad(table, ((0, V_pad - V), (0, 0)))
    tw = pl.pallas_call(
        functools.partial(_tw_tc_kernel, inv_l=1.0 / L),
        out_shape=jax.ShapeDtypeStruct((V_pad, 1), jnp.float32),
    )(table_p, W, b)

    x_flat = x.reshape(-1).astype(jnp.int32)
    out = _make_sc_lookup(B * L, V_pad, B, L, 32, 128)(x_flat, tw.reshape(-1))
    return out.reshape(B, 1)


# speedup vs baseline: 173.6752x; 1.3980x over previous
import functools

import jax
import jax.numpy as jnp
from jax import lax
from jax.experimental import pallas as pl
from jax.experimental.pallas import tpu as pltpu
from jax.experimental.pallas import tpu_sc as plsc

LANES = 16


def _tw_tc_kernel(table_ref, w_ref, b_ref, out_ref, *, inv_l):
    t = table_ref[...]
    w = w_ref[...]
    tw = jnp.dot(t, w, preferred_element_type=jnp.float32)
    out_ref[...] = (tw + b_ref[0]) * inv_l


def _make_sc_lookup(BL_flat, V_pad, B, L, n_workers, chunk_sents):
    sents_per_worker = B // n_workers
    n_chunks = sents_per_worker // chunk_sents
    blocks_per_chunk = chunk_sents // LANES
    mesh = plsc.VectorSubcoreMesh(core_axis_name="c", subcore_axis_name="s")

    @functools.partial(
        pl.kernel,
        mesh=mesh,
        out_type=jax.ShapeDtypeStruct((B,), jnp.float32),
        scratch_types=[
            pltpu.VMEM((chunk_sents * L,), jnp.int32),
            pltpu.VMEM((V_pad,), jnp.float32),
            pltpu.VMEM((sents_per_worker,), jnp.float32),
        ],
        compiler_params=pltpu.CompilerParams(needs_layout_passes=False),
    )
    def sc_lookup(x_hbm, tw_hbm, out_hbm, idx_v, tw_v, out_v):
        n_cores = 2
        wid = lax.axis_index("s") * n_cores + lax.axis_index("c")
        base_s = wid * sents_per_worker

        pltpu.sync_copy(tw_hbm, tw_v)
        iota16 = lax.iota(jnp.int32, LANES)

        for c in range(n_chunks):
            start = (base_s + c * chunk_sents) * L
            pltpu.sync_copy(x_hbm.at[pl.ds(start, chunk_sents * L)], idx_v)
            for blk in range(blocks_per_chunk):
                rowbase = (blk * LANES + iota16) * L

                def body(t, acc, rowbase=rowbase):
                    fidx = rowbase + t
                    xv = plsc.load_gather(idx_v, [fidx])
                    tv = plsc.load_gather(tw_v, [xv])
                    return acc + tv

                acc = lax.fori_loop(0, L, body,
                                    jnp.zeros((LANES,), jnp.float32),
                                    unroll=8)
                pred = 1.0 / (1.0 + jnp.exp(-acc))
                out_v[pl.ds(c * chunk_sents + blk * LANES, LANES)] = pred

        pltpu.sync_copy(out_v, out_hbm.at[pl.ds(base_s, sents_per_worker)])

    return sc_lookup


def kernel(x, table, W, b):
    B, L = x.shape
    V, EMB = table.shape
    V_pad = ((V + 7) // 8) * 8

    table_p = jnp.pad(table, ((0, V_pad - V), (0, 0)))
    tw = pl.pallas_call(
        functools.partial(_tw_tc_kernel, inv_l=1.0 / L),
        out_shape=jax.ShapeDtypeStruct((V_pad, 1), jnp.float32),
    )(table_p, W, b)

    x_flat = x.reshape(-1).astype(jnp.int32)
    out = _make_sc_lookup(B * L, V_pad, B, L, 32, 512)(x_flat, tw.reshape(-1))
    return out.reshape(B, 1)
